# baseline (speedup 1.0000x reference)
import functools

import jax
import jax.numpy as jnp
from jax import lax
from jax.experimental import pallas as pl
from jax.experimental.pallas import tpu as pltpu
from jax.experimental.pallas import tpu_sc as plsc

_B, _C, _H, _W, _K = 64, 2, 512, 512, 500
_NW = 32
_BPW = _B // _NW
_ELEMS = _BPW * _K * _C
_NCHUNK = _ELEMS // 16
_ROWS = _BPW * _C * _H
_PROWS = 64
_NPASS = _ROWS // _PROWS
_KPAD = 512


def _sc_partials(out4, ind, mask, target):
  mesh = plsc.VectorSubcoreMesh(core_axis_name="c", subcore_axis_name="s")

  @functools.partial(
      pl.kernel,
      out_type=jax.ShapeDtypeStruct((2, _NW, 16), jnp.float32),
      mesh=mesh,
      compiler_params=pltpu.CompilerParams(needs_layout_passes=False),
      scratch_types=[
          pltpu.VMEM((_BPW * _K,), jnp.int32),
          pltpu.VMEM((_BPW * _K,), jnp.int32),
          pltpu.VMEM((_ELEMS,), jnp.float32),
          pltpu.VMEM((_ELEMS,), jnp.int32),
          pltpu.VMEM((_ELEMS + 48,), jnp.int32),
          pltpu.VMEM((_NPASS,), jnp.int32),
          pltpu.VMEM((_NPASS,), jnp.int32),
          pltpu.VMEM((_PROWS, _W), jnp.float32),
          pltpu.VMEM((_PROWS, _W), jnp.float32),
          pltpu.VMEM((16,), jnp.float32),
          pltpu.SemaphoreType.DMA,
          pltpu.SemaphoreType.DMA,
      ],
  )
  def k(out4_hbm, ind_hbm, mask_hbm, tgt_hbm, part_hbm,
        ind_v, mask_v, tgt_v, pid_v, bidx_v, counts_v, ptr_v,
        buf0, buf1, stage_v, sem0, sem1):
    out2 = out4_hbm.reshape(_B * _C * _H, _W)
    wid = lax.axis_index("c") * 16 + lax.axis_index("s")
    pltpu.sync_copy(ind_hbm.at[wid], ind_v)
    pltpu.sync_copy(mask_hbm.at[wid], mask_v)
    pltpu.sync_copy(tgt_hbm.at[wid], tgt_v)

    iota = lax.iota(jnp.int32, 16)
    zero = jnp.zeros((16,), jnp.float32)
    izero = jnp.zeros((16,), jnp.int32)
    counts_v[pl.ds(0, 16)] = izero
    counts_v[pl.ds(16, 16)] = izero

    def elem_fields(j):
      p = j >> 1
      ch = j & 1
      bl = (j * 33555) >> 25
      return p, ch, bl

    def build(i, macc):
      j = i * 16 + iota
      p2, ch, bl = elem_fields(j)
      hw = plsc.load_gather(ind_v, [p2])
      mg = plsc.load_gather(mask_v, [p2]).astype(jnp.float32)
      lr = ((bl * _C + ch) << 9) + (hw >> 9)
      pid = lr >> 6
      pid_v[pl.ds(i * 16, 16)] = pid
      cnt, last = plsc.scan_count(pid)
      plsc.addupdate_scatter(counts_v, [pid], cnt, mask=last)
      return macc + mg

    macc = lax.fori_loop(0, _NCHUNK, build, zero)

    c0 = counts_v[pl.ds(0, 16)]
    c1 = counts_v[pl.ds(16, 16)]
    incl0 = plsc.cumsum(c0)
    incl1 = plsc.cumsum(c1)
    excl0 = incl0 - c0
    tot0 = incl0[15]
    g1 = (incl1 - c1) + tot0
    h1 = incl1 + tot0
    ptr_v[pl.ds(0, 16)] = excl0
    ptr_v[pl.ds(16, 16)] = g1

    def place(i, _):
      j = i * 16 + iota
      pid = pid_v[pl.ds(i * 16, 16)]
      cnt, last = plsc.scan_count(pid)
      base = plsc.load_gather(ptr_v, [pid])
      plsc.store_scatter(bidx_v, [base + cnt - 1], j)
      plsc.store_scatter(ptr_v, [pid], base + cnt, mask=last)
      return 0

    lax.fori_loop(0, _NCHUNK, place, 0)

    base0 = pl.multiple_of(wid * _ROWS, _PROWS)
    bufs = (buf0, buf1)
    sems = (sem0, sem1)
    cps = [pltpu.async_copy(out2.at[pl.ds(base0, _PROWS)], buf0, sem0),
           None]

    def sweep(buf, start, end):
      def body(c, a):
        j16 = start + c * 16 + iota
        valid = j16 < end
        eid = plsc.load_gather(bidx_v, [j16 & 2047])
        eid = jnp.minimum(eid & 2047, _ELEMS - 1)
        p2, ch, bl = elem_fields(eid)
        hw = plsc.load_gather(ind_v, [p2])
        mg = plsc.load_gather(mask_v, [p2]).astype(jnp.float32)
        tg = plsc.load_gather(tgt_v, [eid])
        lrow = (((bl * _C + ch) << 9) + (hw >> 9)) & (_PROWS - 1)
        val = plsc.load_gather(buf, [lrow, hw & (_W - 1)])
        return a + jnp.where(valid, jnp.abs(val - tg) * mg, 0.0)
      return body

    acc = zero
    for pss in range(_NPASS):
      cur = pss & 1
      cps[cur].wait()
      if pss + 1 < _NPASS:
        nxt = (pss + 1) & 1
        base = pl.multiple_of(wid * _ROWS + (pss + 1) * _PROWS, _PROWS)
        cps[nxt] = pltpu.async_copy(out2.at[pl.ds(base, _PROWS)],
                                    bufs[nxt], sems[nxt])
      start = excl0[pss] if pss < 16 else g1[pss - 16]
      end = incl0[pss] if pss < 16 else h1[pss - 16]
      nch = (end - start + 15) >> 4
      acc = lax.fori_loop(0, nch, sweep(bufs[cur], start, end), acc)

    stage_v[...] = acc
    pltpu.sync_copy(stage_v, part_hbm.at[0, wid])
    stage_v[...] = macc
    pltpu.sync_copy(stage_v, part_hbm.at[1, wid])

  return k(out4, ind, mask, target)


def _finalize(parts):
  def body(p_ref, o_ref):
    p = p_ref[...]
    o_ref[0, 0] = jnp.sum(p[0]) / (jnp.sum(p[1]) + 0.0001)

  return pl.pallas_call(
      body,
      out_shape=jax.ShapeDtypeStruct((1, 1), jnp.float32),
      out_specs=pl.BlockSpec(memory_space=pltpu.SMEM),
  )(parts)


def kernel(output, mask, ind, target):
  ind2 = ind.reshape(_NW, _BPW * _K)
  mask2 = mask.reshape(_NW, _BPW * _K)
  tgt2 = target.reshape(_NW, _ELEMS)
  parts = _sc_partials(output, ind2, mask2, tgt2)
  return _finalize(parts)[0, 0]


# speedup vs baseline: 1.1986x; 1.1986x over previous
import functools

import jax
import jax.numpy as jnp
from jax import lax
from jax.experimental import pallas as pl
from jax.experimental.pallas import tpu as pltpu
from jax.experimental.pallas import tpu_sc as plsc

_B, _C, _H, _W, _K = 64, 2, 512, 512, 500
_NW = 32
_BPW = _B // _NW
_ELEMS = _BPW * _K * _C
_NCHUNK = _ELEMS // 16
_ROWS = _BPW * _C * _H
_PROWS = 64
_NPASS = _ROWS // _PROWS
_KPAD = 512


def _sc_partials(out4, ind, mask):
  mesh = plsc.VectorSubcoreMesh(core_axis_name="c", subcore_axis_name="s")

  @functools.partial(
      pl.kernel,
      out_type=(jax.ShapeDtypeStruct((_NW, _ELEMS), jnp.float32),
                jax.ShapeDtypeStruct((_NW, _ELEMS), jnp.float32)),
      mesh=mesh,
      compiler_params=pltpu.CompilerParams(needs_layout_passes=False),
      scratch_types=[
          pltpu.VMEM((_BPW * _K,), jnp.int32),
          pltpu.VMEM((_BPW * _K,), jnp.int32),
          pltpu.VMEM((_ELEMS,), jnp.float32),
          pltpu.VMEM((_ELEMS,), jnp.float32),
          pltpu.VMEM((_ELEMS,), jnp.int32),
          pltpu.VMEM((_ELEMS + 48,), jnp.int32),
          pltpu.VMEM((_NPASS,), jnp.int32),
          pltpu.VMEM((_NPASS,), jnp.int32),
          pltpu.VMEM((_PROWS, _W), jnp.float32),
          pltpu.VMEM((_PROWS, _W), jnp.float32),
          pltpu.SemaphoreType.DMA,
          pltpu.SemaphoreType.DMA,
      ],
  )
  def k(out4_hbm, ind_hbm, mask_hbm, predm_hbm, mexp_hbm,
        ind_v, mask_v, pred_v, mgx_v, pid_v, bidx_v, counts_v, ptr_v,
        buf0, buf1, sem0, sem1):
    out2 = out4_hbm.reshape(_B * _C * _H, _W)
    wid = lax.axis_index("c") * 16 + lax.axis_index("s")
    pltpu.sync_copy(ind_hbm.at[wid], ind_v)
    pltpu.sync_copy(mask_hbm.at[wid], mask_v)

    iota = lax.iota(jnp.int32, 16)
    zero = jnp.zeros((16,), jnp.float32)
    izero = jnp.zeros((16,), jnp.int32)
    counts_v[pl.ds(0, 16)] = izero
    counts_v[pl.ds(16, 16)] = izero

    def elem_fields(j):
      p = j >> 1
      ch = j & 1
      bl = (j * 33555) >> 25
      return p, ch, bl

    def build(i, _):
      j = i * 16 + iota
      p2, ch, bl = elem_fields(j)
      hw = plsc.load_gather(ind_v, [p2])
      mg = plsc.load_gather(mask_v, [p2]).astype(jnp.float32)
      lr = ((bl * _C + ch) << 9) + (hw >> 9)
      pid = lr >> 6
      pid_v[pl.ds(i * 16, 16)] = pid
      mgx_v[pl.ds(i * 16, 16)] = mg
      cnt, last = plsc.scan_count(pid)
      plsc.addupdate_scatter(counts_v, [pid], cnt, mask=last)
      return 0

    lax.fori_loop(0, _NCHUNK, build, 0)

    c0 = counts_v[pl.ds(0, 16)]
    c1 = counts_v[pl.ds(16, 16)]
    incl0 = plsc.cumsum(c0)
    incl1 = plsc.cumsum(c1)
    excl0 = incl0 - c0
    tot0 = incl0[15]
    g1 = (incl1 - c1) + tot0
    h1 = incl1 + tot0
    ptr_v[pl.ds(0, 16)] = excl0
    ptr_v[pl.ds(16, 16)] = g1

    def place(i, _):
      j = i * 16 + iota
      pid = pid_v[pl.ds(i * 16, 16)]
      cnt, last = plsc.scan_count(pid)
      base = plsc.load_gather(ptr_v, [pid])
      plsc.store_scatter(bidx_v, [base + cnt - 1], j)
      plsc.store_scatter(ptr_v, [pid], base + cnt, mask=last)
      return 0

    lax.fori_loop(0, _NCHUNK, place, 0)

    base0 = pl.multiple_of(wid * _ROWS, _PROWS)
    bufs = (buf0, buf1)
    sems = (sem0, sem1)
    cps = [pltpu.async_copy(out2.at[pl.ds(base0, _PROWS)], buf0, sem0),
           None]

    def sweep(buf, start, end):
      def body(c, _):
        j16 = start + c * 16 + iota
        valid = j16 < end
        eid = plsc.load_gather(bidx_v, [j16 & 2047])
        eid = jnp.minimum(eid & 2047, _ELEMS - 1)
        p2, ch, bl = elem_fields(eid)
        hw = plsc.load_gather(ind_v, [p2])
        mg = plsc.load_gather(mask_v, [p2]).astype(jnp.float32)
        lrow = (((bl * _C + ch) << 9) + (hw >> 9)) & (_PROWS - 1)
        val = plsc.load_gather(buf, [lrow, hw & (_W - 1)])
        plsc.store_scatter(pred_v, [eid], val * mg, mask=valid)
        return 0
      return body

    for pss in range(_NPASS):
      cur = pss & 1
      cps[cur].wait()
      if pss + 1 < _NPASS:
        nxt = (pss + 1) & 1
        base = pl.multiple_of(wid * _ROWS + (pss + 1) * _PROWS, _PROWS)
        cps[nxt] = pltpu.async_copy(out2.at[pl.ds(base, _PROWS)],
                                    bufs[nxt], sems[nxt])
      start = excl0[pss] if pss < 16 else g1[pss - 16]
      end = incl0[pss] if pss < 16 else h1[pss - 16]
      nch = (end - start + 15) >> 4
      lax.fori_loop(0, nch, sweep(bufs[cur], start, end), 0)

    pltpu.sync_copy(pred_v, predm_hbm.at[wid])
    pltpu.sync_copy(mgx_v, mexp_hbm.at[wid])

  return k(out4, ind, mask)


def _finalize(predm, mexp, tgt2):
  def body(pm_ref, mx_ref, tg_ref, o_ref):
    pm = pm_ref[...]
    mx = mx_ref[...]
    tg = tg_ref[...]
    o_ref[0, 0] = jnp.sum(jnp.abs(pm - tg * mx)) / (jnp.sum(mx) + 0.0001)

  return pl.pallas_call(
      body,
      out_shape=jax.ShapeDtypeStruct((1, 1), jnp.float32),
      out_specs=pl.BlockSpec(memory_space=pltpu.SMEM),
  )(predm, mexp, tgt2)


def kernel(output, mask, ind, target):
  ind2 = ind.reshape(_NW, _BPW * _K)
  mask2 = mask.reshape(_NW, _BPW * _K)
  tgt2 = target.reshape(_NW, _ELEMS)
  predm, mexp = _sc_partials(output, ind2, mask2)
  return _finalize(predm, mexp, tgt2)[0, 0]
